# baseline (speedup 1.0000x reference)
import functools

import jax
import jax.numpy as jnp
from jax import lax
from jax.experimental import pallas as pl
from jax.experimental.pallas import tpu as pltpu
from jax.experimental.pallas import tpu_sc as plsc

N = 10000
E = 320000
D = 128

K = 128
CPW = 80
GSZ = 8
NG = CPW // GSZ
EPW = CPW * K
E_PAD = 32 * EPW
ACC_ROWS = 10112
ZBLOCKS = ACC_ROWS // 128
ORPT = 624


def _sc_segsum_body(u_hbm, src_hbm, dst_hbm, out_hbm,
                    src_v, dst_a, dst_b, buf0, buf1, acc_sh,
                    gsem0, gsem1, ssem0, ssem1, isem):
    c = lax.axis_index("c")
    s = lax.axis_index("s")
    wid = s * 2 + c
    bufs = (buf0, buf1)
    gsems = (gsem0, gsem1)
    ssems = (ssem0, ssem1)

    pltpu.sync_copy(src_hbm.at[wid], src_v)
    pltpu.sync_copy(dst_hbm.at[wid, 0], dst_a)

    def _zb(i, carry):
        buf0[i // 8, pl.ds((i % 8) * 16, 16)] = jnp.zeros((16,), jnp.float32)
        return carry
    lax.fori_loop(0, 128 * 8, _zb, 0)

    for kblk in range(5):
        blk = s + 16 * kblk
        if kblk < 4:
            pltpu.sync_copy(buf0, acc_sh.at[pl.ds(blk * 128, 128)])
        else:
            @pl.when(s < ZBLOCKS - 64)
            def _():
                pltpu.sync_copy(buf0, acc_sh.at[pl.ds(blk * 128, 128)])
    plsc.subcore_barrier()

    pltpu.async_copy(u_hbm.at[src_v.at[0]], buf0, gsem0)
    pltpu.async_copy(u_hbm.at[src_v.at[1]], buf1, gsem1)

    def _gwait(b):
        pltpu.make_async_copy(u_hbm.at[src_v.at[0]], bufs[b], gsems[b]).wait()

    def _swait(b):
        pltpu.make_async_copy(bufs[b], acc_sh.at[dst_a.at[0]],
                              ssems[b]).wait()

    def _outer(i, carry):
        for gpart, dst_x, dst_y in ((0, dst_a, dst_b), (1, dst_b, dst_a)):
            g = i * 2 + gpart
            base = g * GSZ

            @pl.when(g > 0)
            def _():
                pltpu.make_async_copy(dst_hbm.at[wid, 0], dst_x, isem).wait()

            for r in range(GSZ):
                j = base + r
                b = r % 2

                _gwait(b)
                pltpu.async_copy(bufs[b], acc_sh.at[dst_x.at[r]],
                                 ssems[b], add=True)

                if r == 0:
                    @pl.when(j > 0)
                    def _():
                        _swait(1 - b)
                        pltpu.async_copy(u_hbm.at[src_v.at[j + 1]],
                                         bufs[1 - b], gsems[1 - b])

                    @pl.when(g < NG - 1)
                    def _():
                        pltpu.async_copy(dst_hbm.at[wid, g + 1], dst_y, isem)
                else:
                    @pl.when(j < CPW - 1)
                    def _():
                        _swait(1 - b)
                        pltpu.async_copy(u_hbm.at[src_v.at[j + 1]],
                                         bufs[1 - b], gsems[1 - b])
        return carry
    lax.fori_loop(0, NG // 2, _outer, 0)

    _swait(0)
    _swait(1)

    plsc.subcore_barrier()
    pltpu.sync_copy(acc_sh.at[pl.ds(s * ORPT, ORPT)],
                    out_hbm.at[c, pl.ds(s * ORPT, ORPT)])

    @pl.when(s == 15)
    def _():
        tail = 16 * ORPT
        pltpu.sync_copy(acc_sh.at[pl.ds(tail, N - tail)],
                        out_hbm.at[c, pl.ds(tail, N - tail)])


@functools.cache
def _sc_segsum():
    return pl.kernel(
        _sc_segsum_body,
        out_type=jax.ShapeDtypeStruct((2, N, D), jnp.float32),
        mesh=plsc.VectorSubcoreMesh(core_axis_name="c", subcore_axis_name="s"),
        compiler_params=pltpu.CompilerParams(use_tc_tiling_on_sc=False),
        scratch_types=[
            pltpu.VMEM((CPW, K), jnp.int32),
            pltpu.VMEM((GSZ, K), jnp.int32),
            pltpu.VMEM((GSZ, K), jnp.int32),
            pltpu.VMEM((K, D), jnp.float32),
            pltpu.VMEM((K, D), jnp.float32),
            pltpu.VMEM_SHARED((ACC_ROWS, D), jnp.float32),
            pltpu.SemaphoreType.DMA,
            pltpu.SemaphoreType.DMA,
            pltpu.SemaphoreType.DMA,
            pltpu.SemaphoreType.DMA,
            pltpu.SemaphoreType.DMA,
        ],
    )


def _tc1_body(x_ref, h_ref, wa_ref, wb_ref, u_ref):
    u_ref[...] = (
        jnp.dot(x_ref[...], wa_ref[...], preferred_element_type=jnp.float32)
        + jnp.dot(h_ref[...], wb_ref[...], preferred_element_type=jnp.float32)
    )


def _tc2_body(x_ref, s0_ref, s1_ref, wx_ref, b1_ref, w2_ref, b2_ref, o_ref):
    z = (jnp.dot(x_ref[...], wx_ref[...], preferred_element_type=jnp.float32)
         + s0_ref[...] + s1_ref[...] + b1_ref[...])
    hid = jnp.maximum(z, 0.0)
    o_ref[...] = (jnp.dot(hid, w2_ref[...], preferred_element_type=jnp.float32)
                  + b2_ref[...])


_ROWS_BLK = 1000


def kernel(x, h, edge_index, W1, b1, W2, b2):
    wx_t = W1[:, :D].T
    wa_t = W1[:, D:2 * D].T
    wb_t = W1[:, 2 * D:].T
    w2_t = W2.T

    grid = (N // _ROWS_BLK,)
    row_spec = pl.BlockSpec((_ROWS_BLK, D), lambda i: (i, 0))
    full_spec = pl.BlockSpec((D, D), lambda i: (0, 0))
    bias_spec = pl.BlockSpec((1, D), lambda i: (0, 0))

    u = pl.pallas_call(
        _tc1_body,
        grid=grid,
        in_specs=[row_spec, row_spec, full_spec, full_spec],
        out_specs=row_spec,
        out_shape=jax.ShapeDtypeStruct((N, D), jnp.float32),
    )(x, h, wa_t, wb_t)

    pad = E_PAD - E
    src3 = jnp.concatenate(
        [edge_index[0], jnp.zeros((pad,), jnp.int32)]).reshape(32, CPW, K)
    dst4 = jnp.concatenate(
        [edge_index[1], jnp.full((pad,), N, jnp.int32)]).reshape(32, NG, GSZ, K)

    parts = _sc_segsum()(u, src3, dst4)

    out = pl.pallas_call(
        _tc2_body,
        grid=grid,
        in_specs=[row_spec, row_spec, row_spec, full_spec, bias_spec,
                  full_spec, bias_spec],
        out_specs=row_spec,
        out_shape=jax.ShapeDtypeStruct((N, D), jnp.float32),
    )(x, parts[0], parts[1], wx_t, b1.reshape(1, D), w2_t, b2.reshape(1, D))
    return out


# speedup vs baseline: 1.0497x; 1.0497x over previous
import functools

import jax
import jax.numpy as jnp
from jax import lax
from jax.experimental import pallas as pl
from jax.experimental.pallas import tpu as pltpu
from jax.experimental.pallas import tpu_sc as plsc

N = 10000
E = 320000
D = 128

K = 128
CPW = 80
GSZ = 8
NG = CPW // GSZ
EPW = CPW * K
E_PAD = 32 * EPW
ACC_ROWS = 10112
ZBLOCKS = ACC_ROWS // 128
ORPT = 624


def _sc_segsum_body(u_hbm, src_hbm, dst_hbm, out_hbm,
                    src_v, dst_a, dst_b, buf0, buf1, acc_sh,
                    gsem0, gsem1, ssem0, ssem1, isem):
    c = lax.axis_index("c")
    s = lax.axis_index("s")
    wid = s * 2 + c
    bufs = (buf0, buf1)
    gsems = (gsem0, gsem1)
    ssems = (ssem0, ssem1)

    pltpu.sync_copy(src_hbm.at[wid], src_v)
    pltpu.sync_copy(dst_hbm.at[wid, 0], dst_a)

    def _zb(i, carry):
        buf0[i // 8, pl.ds((i % 8) * 16, 16)] = jnp.zeros((16,), jnp.float32)
        return carry
    lax.fori_loop(0, 128 * 8, _zb, 0)

    for kblk in range(5):
        blk = s + 16 * kblk
        if kblk < 4:
            pltpu.sync_copy(buf0, acc_sh.at[pl.ds(blk * 128, 128)])
        else:
            @pl.when(s < ZBLOCKS - 64)
            def _():
                pltpu.sync_copy(buf0, acc_sh.at[pl.ds(blk * 128, 128)])
    plsc.subcore_barrier()

    def _gstart(b, j):
        pltpu.async_copy(u_hbm.at[src_v.at[2 * j]],
                         bufs[b].at[pl.ds(0, 64)], gsems[b])
        pltpu.async_copy(u_hbm.at[src_v.at[2 * j + 1]],
                         bufs[b].at[pl.ds(64, 64)], gsems[b])

    def _gwait(b):
        for _ in range(2):
            pltpu.make_async_copy(u_hbm.at[src_v.at[0]],
                                  bufs[b].at[pl.ds(0, 64)], gsems[b]).wait()

    _gstart(0, 0)
    _gstart(1, 1)

    def _swait(b):
        pltpu.make_async_copy(bufs[b], acc_sh.at[dst_a.at[0]],
                              ssems[b]).wait()

    def _outer(i, carry):
        for gpart, dst_x, dst_y in ((0, dst_a, dst_b), (1, dst_b, dst_a)):
            g = i * 2 + gpart
            base = g * GSZ

            @pl.when(g > 0)
            def _():
                pltpu.make_async_copy(dst_hbm.at[wid, 0], dst_x, isem).wait()

            for r in range(GSZ):
                j = base + r
                b = r % 2

                _gwait(b)
                pltpu.async_copy(bufs[b], acc_sh.at[dst_x.at[r]],
                                 ssems[b], add=True)

                if r == 0:
                    @pl.when(j > 0)
                    def _():
                        _swait(1 - b)
                        _gstart(1 - b, j + 1)

                    @pl.when(g < NG - 1)
                    def _():
                        pltpu.async_copy(dst_hbm.at[wid, g + 1], dst_y, isem)
                else:
                    @pl.when(j < CPW - 1)
                    def _():
                        _swait(1 - b)
                        _gstart(1 - b, j + 1)
        return carry
    lax.fori_loop(0, NG // 2, _outer, 0)

    _swait(0)
    _swait(1)

    plsc.subcore_barrier()
    pltpu.sync_copy(acc_sh.at[pl.ds(s * ORPT, ORPT)],
                    out_hbm.at[c, pl.ds(s * ORPT, ORPT)])

    @pl.when(s == 15)
    def _():
        tail = 16 * ORPT
        pltpu.sync_copy(acc_sh.at[pl.ds(tail, N - tail)],
                        out_hbm.at[c, pl.ds(tail, N - tail)])


@functools.cache
def _sc_segsum():
    return pl.kernel(
        _sc_segsum_body,
        out_type=jax.ShapeDtypeStruct((2, N, D), jnp.float32),
        mesh=plsc.VectorSubcoreMesh(core_axis_name="c", subcore_axis_name="s"),
        compiler_params=pltpu.CompilerParams(use_tc_tiling_on_sc=False),
        scratch_types=[
            pltpu.VMEM((2 * CPW, K // 2), jnp.int32),
            pltpu.VMEM((GSZ, K), jnp.int32),
            pltpu.VMEM((GSZ, K), jnp.int32),
            pltpu.VMEM((K, D), jnp.float32),
            pltpu.VMEM((K, D), jnp.float32),
            pltpu.VMEM_SHARED((ACC_ROWS, D), jnp.float32),
            pltpu.SemaphoreType.DMA,
            pltpu.SemaphoreType.DMA,
            pltpu.SemaphoreType.DMA,
            pltpu.SemaphoreType.DMA,
            pltpu.SemaphoreType.DMA,
        ],
    )


def _tc1_body(x_ref, h_ref, wa_ref, wb_ref, u_ref):
    u_ref[...] = (
        jnp.dot(x_ref[...], wa_ref[...], preferred_element_type=jnp.float32)
        + jnp.dot(h_ref[...], wb_ref[...], preferred_element_type=jnp.float32)
    )


def _tc2_body(x_ref, s0_ref, s1_ref, wx_ref, b1_ref, w2_ref, b2_ref, o_ref):
    z = (jnp.dot(x_ref[...], wx_ref[...], preferred_element_type=jnp.float32)
         + s0_ref[...] + s1_ref[...] + b1_ref[...])
    hid = jnp.maximum(z, 0.0)
    o_ref[...] = (jnp.dot(hid, w2_ref[...], preferred_element_type=jnp.float32)
                  + b2_ref[...])


_ROWS_BLK = 1000


def kernel(x, h, edge_index, W1, b1, W2, b2):
    wx_t = W1[:, :D].T
    wa_t = W1[:, D:2 * D].T
    wb_t = W1[:, 2 * D:].T
    w2_t = W2.T

    grid = (N // _ROWS_BLK,)
    row_spec = pl.BlockSpec((_ROWS_BLK, D), lambda i: (i, 0))
    full_spec = pl.BlockSpec((D, D), lambda i: (0, 0))
    bias_spec = pl.BlockSpec((1, D), lambda i: (0, 0))

    u = pl.pallas_call(
        _tc1_body,
        grid=grid,
        in_specs=[row_spec, row_spec, full_spec, full_spec],
        out_specs=row_spec,
        out_shape=jax.ShapeDtypeStruct((N, D), jnp.float32),
    )(x, h, wa_t, wb_t)

    pad = E_PAD - E
    src3 = jnp.concatenate(
        [edge_index[0], jnp.zeros((pad,), jnp.int32)]).reshape(32, 2 * CPW, K // 2)
    dst4 = jnp.concatenate(
        [edge_index[1], jnp.full((pad,), N, jnp.int32)]).reshape(32, NG, GSZ, K)

    parts = _sc_segsum()(u, src3, dst4)

    out = pl.pallas_call(
        _tc2_body,
        grid=grid,
        in_specs=[row_spec, row_spec, row_spec, full_spec, bias_spec,
                  full_spec, bias_spec],
        out_specs=row_spec,
        out_shape=jax.ShapeDtypeStruct((N, D), jnp.float32),
    )(x, parts[0], parts[1], wx_t, b1.reshape(1, D), w2_t, b2.reshape(1, D))
    return out


# speedup vs baseline: 1.4752x; 1.4054x over previous
import functools

import jax
import jax.numpy as jnp
from jax import lax
from jax.experimental import pallas as pl
from jax.experimental.pallas import tpu as pltpu
from jax.experimental.pallas import tpu_sc as plsc

N = 10000
E = 320000
D = 128
DH = 64

K = 128
CC = 160
GSZ = 8
NG = CC // GSZ
NB = 8
EPW = CC * K
E_PAD = 16 * EPW
ACC_ROWS = 10112
ZBLOCKS = ACC_ROWS // 128
ORPT = 624


def _sc_segsum_body(u_lo_hbm, u_hi_hbm, src_hbm, dst_hbm, out_hbm, *rest):
    src_v, dst_a, dst_b = rest[0], rest[1], rest[2]
    bufs = rest[3:3 + NB]
    acc_sh = rest[3 + NB]
    gsems = rest[4 + NB:4 + 2 * NB]
    ssems = rest[4 + 2 * NB:4 + 3 * NB]
    isem = rest[4 + 3 * NB]

    c = lax.axis_index("c")
    s = lax.axis_index("s")

    pltpu.sync_copy(src_hbm.at[s], src_v)
    pltpu.sync_copy(dst_hbm.at[s, 0], dst_a)

    def _zb(i, carry):
        bufs[0][i // 4, pl.ds((i % 4) * 16, 16)] = jnp.zeros((16,), jnp.float32)
        return carry
    lax.fori_loop(0, 128 * 4, _zb, 0)

    for kblk in range(5):
        blk = s + 16 * kblk
        if kblk < 4:
            pltpu.sync_copy(bufs[0], acc_sh.at[pl.ds(blk * 128, 128)])
        else:
            @pl.when(s < ZBLOCKS - 64)
            def _():
                pltpu.sync_copy(bufs[0], acc_sh.at[pl.ds(blk * 128, 128)])
    plsc.subcore_barrier()

    def _run(u_hbm):
        def _gwait(b):
            pltpu.make_async_copy(u_hbm.at[src_v.at[0]], bufs[b],
                                  gsems[b]).wait()

        def _swait(b):
            pltpu.make_async_copy(bufs[b], acc_sh.at[dst_a.at[0]],
                                  ssems[b]).wait()

        for b in range(NB - 1):
            pltpu.async_copy(u_hbm.at[src_v.at[b]], bufs[b], gsems[b])

        def _outer(i, carry):
            for gpart, dst_x, dst_y in ((0, dst_a, dst_b), (1, dst_b, dst_a)):
                g = i * 2 + gpart
                base = g * GSZ

                @pl.when(g > 0)
                def _():
                    pltpu.make_async_copy(dst_hbm.at[s, 0], dst_x,
                                          isem).wait()

                for r in range(GSZ):
                    j = base + r
                    q = (r + NB - 1) % NB

                    @pl.when(j + NB - 1 < CC)
                    def _():
                        @pl.when(j > 0)
                        def _():
                            _swait(q)
                        pltpu.async_copy(u_hbm.at[src_v.at[j + NB - 1]],
                                         bufs[q], gsems[q])

                    if r == 2:
                        @pl.when(g < NG - 1)
                        def _():
                            pltpu.async_copy(dst_hbm.at[s, g + 1], dst_y,
                                             isem)

                    _gwait(r)
                    pltpu.async_copy(bufs[r], acc_sh.at[dst_x.at[r]],
                                     ssems[r], add=True)
            return carry
        lax.fori_loop(0, NG // 2, _outer, 0)

        for b in range(NB):
            _swait(b)

    @pl.when(c == 0)
    def _():
        _run(u_lo_hbm)

    @pl.when(c == 1)
    def _():
        _run(u_hi_hbm)

    plsc.subcore_barrier()
    pltpu.sync_copy(acc_sh.at[pl.ds(s * ORPT, ORPT)],
                    out_hbm.at[c, pl.ds(s * ORPT, ORPT)])

    @pl.when(s == 15)
    def _():
        tail = 16 * ORPT
        pltpu.sync_copy(acc_sh.at[pl.ds(tail, N - tail)],
                        out_hbm.at[c, pl.ds(tail, N - tail)])


@functools.cache
def _sc_segsum():
    return pl.kernel(
        _sc_segsum_body,
        out_type=jax.ShapeDtypeStruct((2, N, DH), jnp.float32),
        mesh=plsc.VectorSubcoreMesh(core_axis_name="c", subcore_axis_name="s"),
        compiler_params=pltpu.CompilerParams(use_tc_tiling_on_sc=False),
        scratch_types=[
            pltpu.VMEM((CC, K), jnp.int32),
            pltpu.VMEM((GSZ, K), jnp.int32),
            pltpu.VMEM((GSZ, K), jnp.int32),
            *[pltpu.VMEM((K, DH), jnp.float32) for _ in range(NB)],
            pltpu.VMEM_SHARED((ACC_ROWS, DH), jnp.float32),
            *[pltpu.SemaphoreType.DMA for _ in range(2 * NB)],
            pltpu.SemaphoreType.DMA,
        ],
    )


def _tc1_body(x_ref, h_ref, wa_ref, wb_ref, ulo_ref, uhi_ref):
    u = (jnp.dot(x_ref[...], wa_ref[...], preferred_element_type=jnp.float32)
         + jnp.dot(h_ref[...], wb_ref[...], preferred_element_type=jnp.float32))
    ulo_ref[...] = u[:, :DH]
    uhi_ref[...] = u[:, DH:]


def _tc2_body(x_ref, s0_ref, s1_ref, wx_ref, b1_ref, w2_ref, b2_ref, o_ref):
    m1 = jnp.concatenate([s0_ref[...], s1_ref[...]], axis=1)
    z = (jnp.dot(x_ref[...], wx_ref[...], preferred_element_type=jnp.float32)
         + m1 + b1_ref[...])
    hid = jnp.maximum(z, 0.0)
    o_ref[...] = (jnp.dot(hid, w2_ref[...], preferred_element_type=jnp.float32)
                  + b2_ref[...])


_ROWS_BLK = 1000


def kernel(x, h, edge_index, W1, b1, W2, b2):
    wx_t = W1[:, :D].T
    wa_t = W1[:, D:2 * D].T
    wb_t = W1[:, 2 * D:].T
    w2_t = W2.T

    grid = (N // _ROWS_BLK,)
    row_spec = pl.BlockSpec((_ROWS_BLK, D), lambda i: (i, 0))
    half_spec = pl.BlockSpec((_ROWS_BLK, DH), lambda i: (i, 0))
    full_spec = pl.BlockSpec((D, D), lambda i: (0, 0))
    bias_spec = pl.BlockSpec((1, D), lambda i: (0, 0))

    u_lo, u_hi = pl.pallas_call(
        _tc1_body,
        grid=grid,
        in_specs=[row_spec, row_spec, full_spec, full_spec],
        out_specs=[half_spec, half_spec],
        out_shape=[jax.ShapeDtypeStruct((N, DH), jnp.float32),
                   jax.ShapeDtypeStruct((N, DH), jnp.float32)],
    )(x, h, wa_t, wb_t)

    pad = E_PAD - E
    src3 = jnp.concatenate(
        [edge_index[0], jnp.zeros((pad,), jnp.int32)]).reshape(16, CC, K)
    dst4 = jnp.concatenate(
        [edge_index[1], jnp.full((pad,), N, jnp.int32)]).reshape(16, NG, GSZ, K)

    parts = _sc_segsum()(u_lo, u_hi, src3, dst4)

    out = pl.pallas_call(
        _tc2_body,
        grid=grid,
        in_specs=[row_spec, half_spec, half_spec, full_spec, bias_spec,
                  full_spec, bias_spec],
        out_specs=row_spec,
        out_shape=jax.ShapeDtypeStruct((N, D), jnp.float32),
    )(x, parts[0], parts[1], wx_t, b1.reshape(1, D), w2_t, b2.reshape(1, D))
    return out
